# baseline (speedup 1.0000x reference)
import functools

import jax
import jax.numpy as jnp
from jax import lax
from jax.experimental import pallas as pl
from jax.experimental.pallas import tpu as pltpu
from jax.experimental.pallas import tpu_sc as plsc

N = 10000
E = 320000
D = 128
H = 16
WID = 32
NC, NS = 2, 16
NT = NC * NS
EPT = E // NT
CH = 128
K = 80
EPAD = K * CH
NPAD = 10112
RPW = NPAD // NS


def _xwe_body(x_ref, wp_ref, e_ref, o_ref):
    o_ref[...] = (
        jnp.dot(x_ref[...], wp_ref[...], preferred_element_type=jnp.float32)
        + e_ref[...]
    )


_sc_mesh = plsc.VectorSubcoreMesh(core_axis_name="c", subcore_axis_name="s")


@functools.partial(
    pl.kernel,
    out_type=jax.ShapeDtypeStruct((NC, NPAD, WID), jnp.float32),
    mesh=_sc_mesh,
    scratch_types=[
        pltpu.VMEM((K, CH), jnp.int32),
        pltpu.VMEM((K, CH), jnp.int32),
        pltpu.VMEM((CH, WID), jnp.float32),
        pltpu.VMEM((CH, WID), jnp.float32),
        pltpu.VMEM_SHARED((NPAD, WID), jnp.float32),
        pltpu.SemaphoreType.DMA,
        pltpu.SemaphoreType.DMA,
    ],
    compiler_params=pltpu.CompilerParams(use_tc_tiling_on_sc=False),
)
def _edge_scatter(xwe_hbm, src_hbm, dst_hbm, zeros_hbm, out_hbm,
                  src_v, dst_v, rows0, rows1, agg_sh, sem0, sem1):
    c = lax.axis_index("c")
    s = lax.axis_index("s")
    t = c * NS + s
    pltpu.sync_copy(zeros_hbm.at[pl.ds(s * RPW, RPW)],
                    agg_sh.at[pl.ds(s * RPW, RPW)])
    plsc.subcore_barrier()
    pltpu.sync_copy(src_hbm.at[t], src_v)
    pltpu.sync_copy(dst_hbm.at[t], dst_v)

    pltpu.async_copy(xwe_hbm.at[src_v.at[0]], rows0, sem0)
    pltpu.async_copy(xwe_hbm.at[src_v.at[1]], rows1, sem1)

    def chunk2(i, carry):
        j = 2 * i
        pltpu.make_async_copy(xwe_hbm.at[src_v.at[0]], rows0, sem0).wait()
        pltpu.sync_copy(rows0, agg_sh.at[dst_v.at[j]], add=True)

        @pl.when(j + 2 < K)
        def _():
            pltpu.async_copy(xwe_hbm.at[src_v.at[j + 2]], rows0, sem0)

        pltpu.make_async_copy(xwe_hbm.at[src_v.at[1]], rows1, sem1).wait()
        pltpu.sync_copy(rows1, agg_sh.at[dst_v.at[j + 1]], add=True)

        @pl.when(j + 3 < K)
        def _():
            pltpu.async_copy(xwe_hbm.at[src_v.at[j + 3]], rows1, sem1)

        return carry

    lax.fori_loop(0, K // 2, chunk2, 0)
    plsc.subcore_barrier()
    pltpu.sync_copy(agg_sh.at[pl.ds(s * RPW, RPW)],
                    out_hbm.at[c, pl.ds(s * RPW, RPW)])


def _mlp_body(p_ref, xwe_ref, bias_ref, w1_ref, b1_ref, w2_ref, b2_ref, o_ref):
    s = p_ref[0, :N, :] + p_ref[1, :N, :] + xwe_ref[...]
    agg = s[:, :H]
    cnt = s[:, H:H + 1]
    out = agg / jnp.maximum(cnt, 1.0) + bias_ref[...]
    h = jnp.maximum(out, 0.0)
    h = jnp.maximum(
        jnp.dot(h, w1_ref[...], preferred_element_type=jnp.float32)
        + b1_ref[...], 0.0)
    y = (jnp.dot(h, w2_ref[...], preferred_element_type=jnp.float32)
         + b2_ref[...])
    o_ref[...] = jax.nn.sigmoid(y)


def kernel(x, edge_index, W, u, c, bias, W1, b1, W2, b2):
    del u, c
    src = edge_index[0].astype(jnp.int32).reshape(NT, EPT)
    dst = edge_index[1].astype(jnp.int32).reshape(NT, EPT)
    pad_s = jnp.zeros((NT, EPAD - EPT), jnp.int32)
    pad_d = jnp.full((NT, EPAD - EPT), N, jnp.int32)
    srcp = jnp.concatenate([src, pad_s], axis=1).reshape(NT, K, CH)
    dstp = jnp.concatenate([dst, pad_d], axis=1).reshape(NT, K, CH)

    wp = jnp.pad(W, ((0, 0), (0, WID - H)))
    e_row = jnp.zeros((1, WID), jnp.float32).at[0, H].set(1.0)
    xwe = pl.pallas_call(
        _xwe_body,
        out_shape=jax.ShapeDtypeStruct((N, WID), jnp.float32),
    )(x, wp, e_row)

    zeros = jnp.zeros((NPAD, WID), jnp.float32)
    parts = _edge_scatter(xwe, srcp, dstp, zeros)

    y = pl.pallas_call(
        _mlp_body,
        out_shape=jax.ShapeDtypeStruct((N, 1), jnp.float32),
    )(parts, xwe, bias.reshape(1, H), W1, b1.reshape(1, 8),
      W2, b2.reshape(1, 1))
    return y


# speedup vs baseline: 1.1323x; 1.1323x over previous
import functools

import jax
import jax.numpy as jnp
from jax import lax
from jax.experimental import pallas as pl
from jax.experimental.pallas import tpu as pltpu
from jax.experimental.pallas import tpu_sc as plsc

N = 10000
E = 320000
D = 128
H = 16
NC, NS = 2, 16
NT = NC * NS
EPT = E // NT
CH = 128
K = 80
EPAD = K * CH
NPAD = 10112
RPW = NPAD // NS


def _xw_body(x_ref, w_ref, o_ref):
    o_ref[...] = jnp.dot(x_ref[...], w_ref[...],
                         preferred_element_type=jnp.float32)


_sc_mesh = plsc.VectorSubcoreMesh(core_axis_name="c", subcore_axis_name="s")


@functools.partial(
    pl.kernel,
    out_type=[
        jax.ShapeDtypeStruct((NC, NPAD, H), jnp.float32),
        jax.ShapeDtypeStruct((NC, NPAD), jnp.float32),
    ],
    mesh=_sc_mesh,
    scratch_types=[
        pltpu.VMEM((K, CH), jnp.int32),
        pltpu.VMEM((K, CH), jnp.int32),
        pltpu.VMEM((CH, H), jnp.float32),
        pltpu.VMEM((CH, H), jnp.float32),
        pltpu.VMEM((CH,), jnp.float32),
        pltpu.VMEM_SHARED((NPAD, H), jnp.float32),
        pltpu.VMEM_SHARED((NPAD,), jnp.float32),
        pltpu.SemaphoreType.DMA,
        pltpu.SemaphoreType.DMA,
    ],
    compiler_params=pltpu.CompilerParams(use_tc_tiling_on_sc=False),
)
def _edge_scatter(xw_hbm, src_hbm, dst_hbm, zrow_hbm, zcnt_hbm,
                  agg_out, cnt_out,
                  src_v, dst_v, rows0, rows1, ones_v, agg_sh, cnt_sh,
                  sem0, sem1):
    c = lax.axis_index("c")
    s = lax.axis_index("s")
    t = c * NS + s
    pltpu.sync_copy(zrow_hbm.at[pl.ds(s * RPW, RPW)],
                    agg_sh.at[pl.ds(s * RPW, RPW)])
    pltpu.sync_copy(zcnt_hbm.at[pl.ds(s * RPW, RPW)],
                    cnt_sh.at[pl.ds(s * RPW, RPW)])
    for k in range(CH // 16):
        ones_v[pl.ds(k * 16, 16)] = jnp.ones((16,), jnp.float32)
    plsc.subcore_barrier()
    pltpu.sync_copy(src_hbm.at[t], src_v)
    pltpu.sync_copy(dst_hbm.at[t], dst_v)

    pltpu.async_copy(xw_hbm.at[src_v.at[0]], rows0, sem0)
    pltpu.async_copy(xw_hbm.at[src_v.at[1]], rows1, sem1)

    def chunk2(i, carry):
        j = 2 * i
        pltpu.make_async_copy(xw_hbm.at[src_v.at[0]], rows0, sem0).wait()
        pltpu.sync_copy(rows0, agg_sh.at[dst_v.at[j]], add=True)
        pltpu.sync_copy(ones_v, cnt_sh.at[dst_v.at[j]], add=True)

        @pl.when(j + 2 < K)
        def _():
            pltpu.async_copy(xw_hbm.at[src_v.at[j + 2]], rows0, sem0)

        pltpu.make_async_copy(xw_hbm.at[src_v.at[1]], rows1, sem1).wait()
        pltpu.sync_copy(rows1, agg_sh.at[dst_v.at[j + 1]], add=True)
        pltpu.sync_copy(ones_v, cnt_sh.at[dst_v.at[j + 1]], add=True)

        @pl.when(j + 3 < K)
        def _():
            pltpu.async_copy(xw_hbm.at[src_v.at[j + 3]], rows1, sem1)

        return carry

    lax.fori_loop(0, K // 2, chunk2, 0)
    plsc.subcore_barrier()
    pltpu.sync_copy(agg_sh.at[pl.ds(s * RPW, RPW)],
                    agg_out.at[c, pl.ds(s * RPW, RPW)])
    pltpu.sync_copy(cnt_sh.at[pl.ds(s * RPW, RPW)],
                    cnt_out.at[c, pl.ds(s * RPW, RPW)])


def _mlp_body(p_ref, c_ref, xw_ref, bias_ref, w1_ref, b1_ref, w2_ref, b2_ref,
              o_ref):
    agg = p_ref[0, :N, :] + p_ref[1, :N, :] + xw_ref[...]
    cnt = c_ref[0, :N, :] + c_ref[1, :N, :] + 1.0
    out = agg / cnt + bias_ref[...]
    h = jnp.maximum(out, 0.0)
    h = jnp.maximum(
        jnp.dot(h, w1_ref[...], preferred_element_type=jnp.float32)
        + b1_ref[...], 0.0)
    y = (jnp.dot(h, w2_ref[...], preferred_element_type=jnp.float32)
         + b2_ref[...])
    o_ref[...] = jax.nn.sigmoid(y)


def kernel(x, edge_index, W, u, c, bias, W1, b1, W2, b2):
    del u, c
    src = edge_index[0].astype(jnp.int32).reshape(NT, EPT)
    dst = edge_index[1].astype(jnp.int32).reshape(NT, EPT)
    pad_s = jnp.zeros((NT, EPAD - EPT), jnp.int32)
    pad_d = jnp.full((NT, EPAD - EPT), N, jnp.int32)
    srcp = jnp.concatenate([src, pad_s], axis=1).reshape(NT, K, CH)
    dstp = jnp.concatenate([dst, pad_d], axis=1).reshape(NT, K, CH)

    xw = pl.pallas_call(
        _xw_body,
        out_shape=jax.ShapeDtypeStruct((N, H), jnp.float32),
    )(x, W)

    zrow = jnp.zeros((NPAD, H), jnp.float32)
    zcnt = jnp.zeros((NPAD,), jnp.float32)
    parts, cnts = _edge_scatter(xw, srcp, dstp, zrow, zcnt)

    y = pl.pallas_call(
        _mlp_body,
        out_shape=jax.ShapeDtypeStruct((N, 1), jnp.float32),
    )(parts, cnts.reshape(NC, NPAD, 1), xw, bias.reshape(1, H),
      W1, b1.reshape(1, 8), W2, b2.reshape(1, 1))
    return y


# speedup vs baseline: 1.2921x; 1.1412x over previous
import functools

import jax
import jax.numpy as jnp
from jax import lax
from jax.experimental import pallas as pl
from jax.experimental.pallas import tpu as pltpu
from jax.experimental.pallas import tpu_sc as plsc

N = 10000
E = 320000
D = 128
H = 16
NC, NS = 2, 16
NT = NC * NS
EPT = E // NT
CH = 128
K = 80
EPAD = K * CH
NPAD = 10112
RPW = NPAD // NS


def _xw_body(x_ref, w_ref, o_ref):
    o_ref[...] = jnp.dot(x_ref[...], w_ref[...],
                         preferred_element_type=jnp.float32)


_sc_mesh = plsc.VectorSubcoreMesh(core_axis_name="c", subcore_axis_name="s")


@functools.partial(
    pl.kernel,
    out_type=[
        jax.ShapeDtypeStruct((NC, NPAD, H), jnp.float32),
        jax.ShapeDtypeStruct((NC, NPAD), jnp.float32),
    ],
    mesh=_sc_mesh,
    scratch_types=[
        pltpu.VMEM((K, CH), jnp.int32),
        pltpu.VMEM((K, CH), jnp.int32),
        pltpu.VMEM((8, CH, H), jnp.float32),
        pltpu.VMEM((CH,), jnp.float32),
        pltpu.VMEM_SHARED((NPAD, H), jnp.float32),
        pltpu.VMEM_SHARED((NPAD,), jnp.float32),
        pltpu.SemaphoreType.DMA((8,)),
        pltpu.SemaphoreType.DMA((8,)),
        pltpu.SemaphoreType.DMA,
    ],
    compiler_params=pltpu.CompilerParams(use_tc_tiling_on_sc=False),
)
def _edge_scatter(xw_hbm, src_hbm, dst_hbm, zrow_hbm, zcnt_hbm,
                  agg_out, cnt_out,
                  src_v, dst_v, rows_v, ones_v, agg_sh, cnt_sh,
                  gsem, ssem, csem):
    c = lax.axis_index("c")
    s = lax.axis_index("s")
    t = c * NS + s
    pltpu.sync_copy(zrow_hbm.at[pl.ds(s * RPW, RPW)],
                    agg_sh.at[pl.ds(s * RPW, RPW)])
    pltpu.sync_copy(zcnt_hbm.at[pl.ds(s * RPW, RPW)],
                    cnt_sh.at[pl.ds(s * RPW, RPW)])
    for k in range(CH // 16):
        ones_v[pl.ds(k * 16, 16)] = jnp.ones((16,), jnp.float32)
    plsc.subcore_barrier()
    pltpu.sync_copy(src_hbm.at[t], src_v)
    pltpu.sync_copy(dst_hbm.at[t], dst_v)

    def start_gather(j, b):
        pltpu.async_copy(xw_hbm.at[src_v.at[j]], rows_v.at[b], gsem.at[b])

    def wait_gather(b):
        pltpu.make_async_copy(xw_hbm.at[src_v.at[0]], rows_v.at[b],
                              gsem.at[b]).wait()

    def start_scatters(j, b):
        pltpu.async_copy(rows_v.at[b], agg_sh.at[dst_v.at[j]], ssem.at[b],
                         add=True)
        pltpu.async_copy(ones_v, cnt_sh.at[dst_v.at[j]], csem, add=True)

    def wait_scatter(b):
        pltpu.make_async_copy(rows_v.at[b], agg_sh.at[dst_v.at[0]],
                              ssem.at[b]).wait()

    def chunk_step(j, b, do_wait_scatter, do_start_gather):
        wait_gather(b)
        start_scatters(j, b)
        b4 = (b + 4) % 8
        if do_wait_scatter:
            wait_scatter(b4)
        if do_start_gather:
            start_gather(j + 4, b4)

    for b in range(4):
        start_gather(b, b)
    for b in range(8):
        chunk_step(b, b, do_wait_scatter=(b >= 4), do_start_gather=True)

    def block(i, carry):
        j0 = 8 * i
        for b in range(8):
            chunk_step(j0 + b, b, True, True)
        return carry

    lax.fori_loop(1, K // 8 - 1, block, 0)
    for b in range(8):
        chunk_step(K - 8 + b, b, True, do_start_gather=(b < 4))
    for b in range(4, 8):
        wait_scatter(b)

    def drain_cnt(j, carry):
        pltpu.make_async_copy(ones_v, cnt_sh.at[dst_v.at[0]], csem).wait()
        return carry

    lax.fori_loop(0, K, drain_cnt, 0)
    plsc.subcore_barrier()
    pltpu.sync_copy(agg_sh.at[pl.ds(s * RPW, RPW)],
                    agg_out.at[c, pl.ds(s * RPW, RPW)])
    pltpu.sync_copy(cnt_sh.at[pl.ds(s * RPW, RPW)],
                    cnt_out.at[c, pl.ds(s * RPW, RPW)])


def _mlp_body(p_ref, c_ref, xw_ref, bias_ref, w1_ref, b1_ref, w2_ref, b2_ref,
              o_ref):
    agg = p_ref[0, :N, :] + p_ref[1, :N, :] + xw_ref[...]
    cnt = c_ref[0, :N, :] + c_ref[1, :N, :] + 1.0
    out = agg / cnt + bias_ref[...]
    h = jnp.maximum(out, 0.0)
    h = jnp.maximum(
        jnp.dot(h, w1_ref[...], preferred_element_type=jnp.float32)
        + b1_ref[...], 0.0)
    y = (jnp.dot(h, w2_ref[...], preferred_element_type=jnp.float32)
         + b2_ref[...])
    o_ref[...] = jax.nn.sigmoid(y)


def kernel(x, edge_index, W, u, c, bias, W1, b1, W2, b2):
    del u, c
    src = edge_index[0].astype(jnp.int32).reshape(NT, EPT)
    dst = edge_index[1].astype(jnp.int32).reshape(NT, EPT)
    pad_s = jnp.zeros((NT, EPAD - EPT), jnp.int32)
    pad_d = jnp.full((NT, EPAD - EPT), N, jnp.int32)
    srcp = jnp.concatenate([src, pad_s], axis=1).reshape(NT, K, CH)
    dstp = jnp.concatenate([dst, pad_d], axis=1).reshape(NT, K, CH)

    xw = pl.pallas_call(
        _xw_body,
        out_shape=jax.ShapeDtypeStruct((N, H), jnp.float32),
    )(x, W)

    zrow = jnp.zeros((NPAD, H), jnp.float32)
    zcnt = jnp.zeros((NPAD,), jnp.float32)
    parts, cnts = _edge_scatter(xw, srcp, dstp, zrow, zcnt)

    y = pl.pallas_call(
        _mlp_body,
        out_shape=jax.ShapeDtypeStruct((N, 1), jnp.float32),
    )(parts, cnts.reshape(NC, NPAD, 1), xw, bias.reshape(1, H),
      W1, b1.reshape(1, 8), W2, b2.reshape(1, 1))
    return y


# speedup vs baseline: 1.3238x; 1.0245x over previous
import functools

import jax
import jax.numpy as jnp
from jax import lax
from jax.experimental import pallas as pl
from jax.experimental.pallas import tpu as pltpu
from jax.experimental.pallas import tpu_sc as plsc

N = 10000
E = 320000
D = 128
H = 16
NC, NS = 2, 16
NT = NC * NS
EPT = E // NT
CH = 128
SB = 8
SCH = SB * CH
KB = 10
K = KB * SB
EPAD = K * CH
NPAD = 10112
RPW = NPAD // NS


def _xw_body(x_ref, w_ref, o_ref):
    o_ref[...] = jnp.dot(x_ref[...], w_ref[...],
                         preferred_element_type=jnp.float32)


_sc_mesh = plsc.VectorSubcoreMesh(core_axis_name="c", subcore_axis_name="s")


@functools.partial(
    pl.kernel,
    out_type=[
        jax.ShapeDtypeStruct((NC, NPAD, H), jnp.float32),
        jax.ShapeDtypeStruct((NC, NPAD), jnp.float32),
    ],
    mesh=_sc_mesh,
    scratch_types=[
        pltpu.VMEM((EPAD,), jnp.int32),
        pltpu.VMEM((EPAD,), jnp.int32),
        pltpu.VMEM((4, SB * CH, H), jnp.float32),
        pltpu.VMEM((SB * CH,), jnp.float32),
        pltpu.VMEM_SHARED((NPAD, H), jnp.float32),
        pltpu.VMEM_SHARED((NPAD,), jnp.float32),
        pltpu.SemaphoreType.DMA((4,)),
        pltpu.SemaphoreType.DMA((4,)),
        pltpu.SemaphoreType.DMA,
    ],
    compiler_params=pltpu.CompilerParams(use_tc_tiling_on_sc=False),
)
def _edge_scatter(xw_hbm, src_hbm, dst_hbm, zrow_hbm, zcnt_hbm,
                  agg_out, cnt_out,
                  src_v, dst_v, rows_v, ones_v, agg_sh, cnt_sh,
                  gsem, ssem, csem):
    c = lax.axis_index("c")
    s = lax.axis_index("s")
    t = c * NS + s
    pltpu.sync_copy(zrow_hbm.at[pl.ds(s * RPW, RPW)],
                    agg_sh.at[pl.ds(s * RPW, RPW)])
    pltpu.sync_copy(zcnt_hbm.at[pl.ds(s * RPW, RPW)],
                    cnt_sh.at[pl.ds(s * RPW, RPW)])
    for k in range(SB * CH // 16):
        ones_v[pl.ds(k * 16, 16)] = jnp.ones((16,), jnp.float32)
    plsc.subcore_barrier()
    pltpu.sync_copy(src_hbm.at[t], src_v)
    pltpu.sync_copy(dst_hbm.at[t], dst_v)

    def start_gather(g, b):
        pltpu.async_copy(xw_hbm.at[src_v.at[pl.ds(g * SCH, SCH)]],
                         rows_v.at[b], gsem.at[b])

    def wait_gather(b):
        pltpu.make_async_copy(xw_hbm.at[src_v.at[pl.ds(0, SCH)]],
                              rows_v.at[b], gsem.at[b]).wait()

    def start_scatters(g, b):
        pltpu.async_copy(rows_v.at[b], agg_sh.at[dst_v.at[pl.ds(g * SCH, SCH)]],
                         ssem.at[b], add=True)
        pltpu.async_copy(ones_v, cnt_sh.at[dst_v.at[pl.ds(g * SCH, SCH)]],
                         csem, add=True)

    def wait_scatter(b):
        pltpu.make_async_copy(rows_v.at[b], agg_sh.at[dst_v.at[pl.ds(0, SCH)]],
                              ssem.at[b]).wait()

    for g in range(3):
        start_gather(g, g)
    for g in range(KB):
        b = g % 4
        wait_gather(b)
        start_scatters(g, b)
        nxt = g + 3
        if nxt < KB:
            bn = nxt % 4
            if nxt >= 4:
                wait_scatter(bn)
            start_gather(nxt, bn)
    for g in range(KB - 4, KB):
        wait_scatter(g % 4)
    for g in range(KB):
        pltpu.make_async_copy(ones_v, cnt_sh.at[dst_v.at[pl.ds(0, SCH)]],
                              csem).wait()
    plsc.subcore_barrier()
    pltpu.sync_copy(agg_sh.at[pl.ds(s * RPW, RPW)],
                    agg_out.at[c, pl.ds(s * RPW, RPW)])
    pltpu.sync_copy(cnt_sh.at[pl.ds(s * RPW, RPW)],
                    cnt_out.at[c, pl.ds(s * RPW, RPW)])


def _mlp_body(p_ref, c_ref, xw_ref, bias_ref, w1_ref, b1_ref, w2_ref, b2_ref,
              o_ref):
    agg = p_ref[0, :N, :] + p_ref[1, :N, :] + xw_ref[...]
    cnt = c_ref[0, :N, :] + c_ref[1, :N, :] + 1.0
    out = agg / cnt + bias_ref[...]
    h = jnp.maximum(out, 0.0)
    h = jnp.maximum(
        jnp.dot(h, w1_ref[...], preferred_element_type=jnp.float32)
        + b1_ref[...], 0.0)
    y = (jnp.dot(h, w2_ref[...], preferred_element_type=jnp.float32)
         + b2_ref[...])
    o_ref[...] = jax.nn.sigmoid(y)


def kernel(x, edge_index, W, u, c, bias, W1, b1, W2, b2):
    del u, c
    src = edge_index[0].astype(jnp.int32).reshape(NT, EPT)
    dst = edge_index[1].astype(jnp.int32).reshape(NT, EPT)
    pad_s = jnp.zeros((NT, EPAD - EPT), jnp.int32)
    pad_d = jnp.full((NT, EPAD - EPT), N, jnp.int32)
    srcp = jnp.concatenate([src, pad_s], axis=1).reshape(NT, EPAD)
    dstp = jnp.concatenate([dst, pad_d], axis=1).reshape(NT, EPAD)

    xw = pl.pallas_call(
        _xw_body,
        out_shape=jax.ShapeDtypeStruct((N, H), jnp.float32),
    )(x, W)

    zrow = jnp.zeros((NPAD, H), jnp.float32)
    zcnt = jnp.zeros((NPAD,), jnp.float32)
    parts, cnts = _edge_scatter(xw, srcp, dstp, zrow, zcnt)

    y = pl.pallas_call(
        _mlp_body,
        out_shape=jax.ShapeDtypeStruct((N, 1), jnp.float32),
    )(parts, cnts.reshape(NC, NPAD, 1), xw, bias.reshape(1, H),
      W1, b1.reshape(1, 8), W2, b2.reshape(1, 1))
    return y


# speedup vs baseline: 1.4851x; 1.1219x over previous
import functools

import jax
import jax.numpy as jnp
from jax import lax
from jax.experimental import pallas as pl
from jax.experimental.pallas import tpu as pltpu
from jax.experimental.pallas import tpu_sc as plsc

N = 10000
E = 320000
D = 128
H = 16
NC, NS = 2, 16
NT = NC * NS
SCH = 512
KB = 20
EPAD = KB * SCH
RING = 8
LOOK = 3
NPAD = 10112
RPW = NPAD // NS
PK = 128 // H
NR = N * H // 128
NRP = NPAD * H // 128


def _xw_body(x_ref, w_ref, o_ref):
    o_ref[...] = jnp.dot(x_ref[...], w_ref[...],
                         preferred_element_type=jnp.float32)


_sc_mesh = plsc.VectorSubcoreMesh(core_axis_name="c", subcore_axis_name="s")


@functools.partial(
    pl.kernel,
    out_type=[
        jax.ShapeDtypeStruct((NC, NPAD, H), jnp.float32),
        jax.ShapeDtypeStruct((NC, NPAD, H), jnp.float32),
    ],
    mesh=_sc_mesh,
    scratch_types=[
        pltpu.VMEM((EPAD,), jnp.int32),
        pltpu.VMEM((EPAD,), jnp.int32),
        pltpu.VMEM((RING, SCH, H), jnp.float32),
        pltpu.VMEM((SCH, H), jnp.float32),
        pltpu.VMEM_SHARED((NPAD, H), jnp.float32),
        pltpu.VMEM_SHARED((NPAD, H), jnp.float32),
        pltpu.SemaphoreType.DMA((RING,)),
        pltpu.SemaphoreType.DMA((RING,)),
        pltpu.SemaphoreType.DMA,
    ],
    compiler_params=pltpu.CompilerParams(use_tc_tiling_on_sc=False),
)
def _edge_scatter(xw_hbm, src_hbm, dst_hbm, zrow_hbm,
                  agg_out, cnt_out,
                  src_v, dst_v, rows_v, ones_v,
                  agg_sh, cnt_sh, gsem, ssem, csem):
    c = lax.axis_index("c")
    s = lax.axis_index("s")
    t = c * NS + s
    pltpu.sync_copy(zrow_hbm.at[pl.ds(s * RPW, RPW)],
                    agg_sh.at[pl.ds(s * RPW, RPW)])
    pltpu.sync_copy(zrow_hbm.at[pl.ds(s * RPW, RPW)],
                    cnt_sh.at[pl.ds(s * RPW, RPW)])
    def fill_ones(r, carry):
        ones_v[r] = jnp.ones((H,), jnp.float32)
        return carry

    lax.fori_loop(0, SCH, fill_ones, 0)
    plsc.subcore_barrier()
    pltpu.sync_copy(src_hbm.at[t], src_v)
    pltpu.sync_copy(dst_hbm.at[t], dst_v)

    def start_gather(g, b):
        pltpu.async_copy(xw_hbm.at[src_v.at[pl.ds(g * SCH, SCH)]],
                         rows_v.at[b], gsem.at[b])

    def wait_gather(b):
        pltpu.make_async_copy(xw_hbm.at[src_v.at[pl.ds(0, SCH)]],
                              rows_v.at[b], gsem.at[b]).wait()

    def start_scatters(g, b):
        pltpu.async_copy(rows_v.at[b], agg_sh.at[dst_v.at[pl.ds(g * SCH, SCH)]],
                         ssem.at[b], add=True)
        pltpu.async_copy(ones_v, cnt_sh.at[dst_v.at[pl.ds(g * SCH, SCH)]],
                         csem, add=True)

    def wait_scatter(b):
        pltpu.make_async_copy(rows_v.at[b], agg_sh.at[dst_v.at[pl.ds(0, SCH)]],
                              ssem.at[b]).wait()

    for g in range(LOOK):
        start_gather(g, g)
    for g in range(KB):
        b = g % RING
        wait_gather(b)
        start_scatters(g, b)
        nxt = g + LOOK
        if nxt < KB:
            bn = nxt % RING
            if nxt >= RING:
                wait_scatter(bn)
            start_gather(nxt, bn)
    for g in range(KB - RING, KB):
        wait_scatter(g % RING)
    for g in range(KB):
        pltpu.make_async_copy(ones_v, cnt_sh.at[dst_v.at[pl.ds(0, SCH)]],
                              csem).wait()
    plsc.subcore_barrier()
    pltpu.sync_copy(agg_sh.at[pl.ds(s * RPW, RPW)],
                    agg_out.at[c, pl.ds(s * RPW, RPW)])
    pltpu.sync_copy(cnt_sh.at[pl.ds(s * RPW, RPW)],
                    cnt_out.at[c, pl.ds(s * RPW, RPW)])


def _mlp_body(p_ref, c_ref, xw_ref, bias_ref, w1_ref, b1_ref, w2_ref, b2_ref,
              o_ref):
    s = p_ref[0, :NR, :] + p_ref[1, :NR, :] + xw_ref[...]
    cnt = c_ref[0, :NR, :] + c_ref[1, :NR, :] + 1.0
    h = jnp.maximum(s / cnt + bias_ref[...], 0.0)
    h = jnp.maximum(
        jnp.dot(h, w1_ref[...], preferred_element_type=jnp.float32)
        + b1_ref[...], 0.0)
    y = (jnp.dot(h, w2_ref[...], preferred_element_type=jnp.float32)
         + b2_ref[...])
    o_ref[...] = jax.nn.sigmoid(y)


def _blockdiag(m, k):
    r, ccol = m.shape
    out = jnp.zeros((k, r, k, ccol), m.dtype)
    out = out.at[jnp.arange(k), :, jnp.arange(k), :].set(m)
    return out.reshape(k * r, k * ccol)


def kernel(x, edge_index, W, u, c, bias, W1, b1, W2, b2):
    del u, c
    srcp = jnp.pad(edge_index[0].astype(jnp.int32),
                   (0, NT * EPAD - E)).reshape(NT, EPAD)
    dstp = jnp.pad(edge_index[1].astype(jnp.int32), (0, NT * EPAD - E),
                   constant_values=N).reshape(NT, EPAD)

    xw8 = pl.pallas_call(
        _xw_body,
        out_shape=jax.ShapeDtypeStruct((NR, 128), jnp.float32),
    )(x.reshape(NR, PK * D), _blockdiag(W, PK))

    zrow = jnp.zeros((NPAD, H), jnp.float32)
    parts, cnts = _edge_scatter(xw8.reshape(N, H), srcp, dstp, zrow)

    y8 = pl.pallas_call(
        _mlp_body,
        out_shape=jax.ShapeDtypeStruct((NR, PK), jnp.float32),
    )(parts.reshape(NC, NRP, 128), cnts.reshape(NC, NRP, 128), xw8,
      jnp.tile(bias, PK).reshape(1, PK * H), _blockdiag(W1, PK),
      jnp.tile(b1, PK).reshape(1, PK * 8), _blockdiag(W2, PK),
      b2.reshape(1, 1))
    return y8.reshape(N, 1)


# speedup vs baseline: 1.6849x; 1.1346x over previous
import functools

import jax
import jax.numpy as jnp
from jax import lax
from jax.experimental import pallas as pl
from jax.experimental.pallas import tpu as pltpu
from jax.experimental.pallas import tpu_sc as plsc

N = 10000
E = 320000
D = 128
H = 16
NC, NS = 2, 16
NT = NC * NS
SCH = 512
KB = 20
EPAD = KB * SCH
RING = 8
LOOK = 3
NPAD = 10112
RPW = NPAD // NS
PK = 128 // H
NR = N * H // 128
NRP = NPAD * H // 128
NTRASH = NPAD - N


def _xw_body(x3_ref, b_ref, o_ref):
    acc = jnp.dot(x3_ref[:, 0, :], b_ref[0],
                  preferred_element_type=jnp.float32)
    for a in range(1, PK):
        acc += jnp.dot(x3_ref[:, a, :], b_ref[a],
                       preferred_element_type=jnp.float32)
    o_ref[...] = acc


_sc_mesh = plsc.VectorSubcoreMesh(core_axis_name="c", subcore_axis_name="s")


@functools.partial(
    pl.kernel,
    out_type=[
        jax.ShapeDtypeStruct((NC, NPAD, H), jnp.float32),
        jax.ShapeDtypeStruct((NC, NPAD, H), jnp.float32),
    ],
    mesh=_sc_mesh,
    scratch_types=[
        pltpu.VMEM((EPAD,), jnp.int32),
        pltpu.VMEM((EPAD,), jnp.int32),
        pltpu.VMEM((RING, SCH, H), jnp.float32),
        pltpu.VMEM((SCH, H), jnp.float32),
        pltpu.VMEM_SHARED((NPAD, H), jnp.float32),
        pltpu.VMEM_SHARED((NPAD, H), jnp.float32),
        pltpu.SemaphoreType.DMA((RING,)),
        pltpu.SemaphoreType.DMA((RING,)),
        pltpu.SemaphoreType.DMA,
    ],
    compiler_params=pltpu.CompilerParams(use_tc_tiling_on_sc=False),
)
def _edge_scatter(xw_hbm, src_hbm, dst_hbm, zrow_hbm,
                  agg_out, cnt_out,
                  src_v, dst_v, rows_v, ones_v,
                  agg_sh, cnt_sh, gsem, ssem, csem):
    c = lax.axis_index("c")
    s = lax.axis_index("s")
    t = c * NS + s
    lastw = N - (NS - 1) * RPW

    @pl.when(jnp.logical_and(c == 0, s < NS - 1))
    def _():
        pltpu.sync_copy(xw_hbm.at[pl.ds(s * RPW, RPW)],
                        agg_sh.at[pl.ds(s * RPW, RPW)])

    @pl.when(jnp.logical_and(c == 0, s == NS - 1))
    def _():
        pltpu.sync_copy(xw_hbm.at[pl.ds((NS - 1) * RPW, lastw)],
                        agg_sh.at[pl.ds((NS - 1) * RPW, lastw)])
        pltpu.sync_copy(zrow_hbm.at[pl.ds(0, NTRASH)],
                        agg_sh.at[pl.ds(N, NTRASH)])

    @pl.when(c == 1)
    def _():
        pltpu.sync_copy(zrow_hbm.at[pl.ds(s * RPW, RPW)],
                        agg_sh.at[pl.ds(s * RPW, RPW)])

    pltpu.sync_copy(zrow_hbm.at[pl.ds(s * RPW, RPW)],
                    cnt_sh.at[pl.ds(s * RPW, RPW)])

    def fill_ones(r, carry):
        ones_v[r] = jnp.ones((H,), jnp.float32)
        return carry

    lax.fori_loop(0, SCH, fill_ones, 0)
    plsc.subcore_barrier()
    pltpu.sync_copy(src_hbm.at[pl.ds(t * EPAD, EPAD)], src_v)
    pltpu.sync_copy(dst_hbm.at[pl.ds(t * EPAD, EPAD)], dst_v)

    def start_gather(g, b):
        pltpu.async_copy(xw_hbm.at[src_v.at[pl.ds(g * SCH, SCH)]],
                         rows_v.at[b], gsem.at[b])

    def wait_gather(b):
        pltpu.make_async_copy(xw_hbm.at[src_v.at[pl.ds(0, SCH)]],
                              rows_v.at[b], gsem.at[b]).wait()

    def start_scatters(g, b):
        pltpu.async_copy(rows_v.at[b], agg_sh.at[dst_v.at[pl.ds(g * SCH, SCH)]],
                         ssem.at[b], add=True)
        pltpu.async_copy(ones_v, cnt_sh.at[dst_v.at[pl.ds(g * SCH, SCH)]],
                         csem, add=True)

    def wait_scatter(b):
        pltpu.make_async_copy(rows_v.at[b], agg_sh.at[dst_v.at[pl.ds(0, SCH)]],
                              ssem.at[b]).wait()

    for g in range(LOOK):
        start_gather(g, g)
    for g in range(KB):
        b = g % RING
        wait_gather(b)
        start_scatters(g, b)
        nxt = g + LOOK
        if nxt < KB:
            bn = nxt % RING
            if nxt >= RING:
                wait_scatter(bn)
            start_gather(nxt, bn)
    for g in range(KB - RING, KB):
        wait_scatter(g % RING)
    for g in range(KB):
        pltpu.make_async_copy(ones_v, cnt_sh.at[dst_v.at[pl.ds(0, SCH)]],
                              csem).wait()
    plsc.subcore_barrier()
    pltpu.sync_copy(agg_sh.at[pl.ds(s * RPW, RPW)],
                    agg_out.at[c, pl.ds(s * RPW, RPW)])
    pltpu.sync_copy(cnt_sh.at[pl.ds(s * RPW, RPW)],
                    cnt_out.at[c, pl.ds(s * RPW, RPW)])


def _mlp_body(p_ref, c_ref, bias_ref, w1_ref, b1_ref, w2_ref, b2_ref, o_ref):
    s = p_ref[0, :NR, :] + p_ref[1, :NR, :]
    cnt = c_ref[0, :NR, :] + c_ref[1, :NR, :] + 1.0
    h = jnp.maximum(s / cnt + bias_ref[...], 0.0)
    h = jnp.maximum(
        jnp.dot(h, w1_ref[...], preferred_element_type=jnp.float32)
        + b1_ref[...], 0.0)
    y = (jnp.dot(h, w2_ref[...], preferred_element_type=jnp.float32)
         + b2_ref[...])
    o_ref[...] = jax.nn.sigmoid(y)


def _blockdiag(m, k):
    r, ccol = m.shape
    out = jnp.zeros((k, r, k, ccol), m.dtype)
    out = out.at[jnp.arange(k), :, jnp.arange(k), :].set(m)
    return out.reshape(k * r, k * ccol)


def kernel(x, edge_index, W, u, c, bias, W1, b1, W2, b2):
    del u, c
    npad = NT * EPAD - E
    srcp = jnp.pad(edge_index[0].astype(jnp.int32), (0, npad))
    dstp = jnp.concatenate([
        edge_index[1].astype(jnp.int32),
        N + (jnp.arange(npad, dtype=jnp.int32) % NTRASH)])

    b4 = jnp.stack([jnp.pad(W, ((0, 0), (a * H, 128 - (a + 1) * H)))
                    for a in range(PK)])
    xw8 = pl.pallas_call(
        _xw_body,
        out_shape=jax.ShapeDtypeStruct((NR, 128), jnp.float32),
    )(x.reshape(NR, PK, D), b4)

    zrow = jnp.zeros((NPAD, H), jnp.float32)
    parts, cnts = _edge_scatter(xw8.reshape(N, H), srcp, dstp, zrow)

    y8 = pl.pallas_call(
        _mlp_body,
        out_shape=jax.ShapeDtypeStruct((NR, PK), jnp.float32),
    )(parts.reshape(NC, NRP, 128), cnts.reshape(NC, NRP, 128),
      jnp.tile(bias, PK).reshape(1, PK * H), _blockdiag(W1, PK),
      jnp.tile(b1, PK).reshape(1, PK * 8), _blockdiag(W2, PK),
      b2.reshape(1, 1))
    return y8.reshape(N, 1)


# speedup vs baseline: 1.9292x; 1.1450x over previous
import functools

import jax
import jax.numpy as jnp
from jax import lax
from jax.experimental import pallas as pl
from jax.experimental.pallas import tpu as pltpu
from jax.experimental.pallas import tpu_sc as plsc

N = 10000
E = 320000
D = 128
H = 16
NC, NS = 2, 16
NT = NC * NS
SCH = 512
KB = 20
EPAD = KB * SCH
RING = 8
LOOK = 3
EPT = E // NT
PADT = EPAD - EPT
NPAD = 10112
RPW = NPAD // NS
PK = 128 // H
NR = N * H // 128
NRP = NPAD * H // 128
NTRASH = NPAD - N


def _xw_body(x3_ref, b_ref, o_ref):
    acc = jnp.dot(x3_ref[:, 0, :], b_ref[0],
                  preferred_element_type=jnp.float32)
    for a in range(1, PK):
        acc += jnp.dot(x3_ref[:, a, :], b_ref[a],
                       preferred_element_type=jnp.float32)
    o_ref[...] = acc


_sc_mesh = plsc.VectorSubcoreMesh(core_axis_name="c", subcore_axis_name="s")


@functools.partial(
    pl.kernel,
    out_type=[
        jax.ShapeDtypeStruct((NC, NPAD, H), jnp.float32),
        jax.ShapeDtypeStruct((NC, NPAD, H), jnp.float32),
    ],
    mesh=_sc_mesh,
    scratch_types=[
        pltpu.VMEM((EPAD,), jnp.int32),
        pltpu.VMEM((EPAD,), jnp.int32),
        pltpu.VMEM((RING, SCH, H), jnp.float32),
        pltpu.VMEM((SCH, H), jnp.float32),
        pltpu.VMEM_SHARED((NPAD, H), jnp.float32),
        pltpu.VMEM_SHARED((NPAD, H), jnp.float32),
        pltpu.SemaphoreType.DMA((RING,)),
        pltpu.SemaphoreType.DMA((RING,)),
        pltpu.SemaphoreType.DMA,
    ],
    compiler_params=pltpu.CompilerParams(use_tc_tiling_on_sc=False),
)
def _edge_scatter(xw_hbm, ei_hbm, psrc_hbm, pdst_hbm, zrow_hbm,
                  agg_out, cnt_out,
                  src_v, dst_v, rows_v, ones_v,
                  agg_sh, cnt_sh, gsem, ssem, csem):
    c = lax.axis_index("c")
    s = lax.axis_index("s")
    t = c * NS + s
    lastw = N - (NS - 1) * RPW

    @pl.when(jnp.logical_and(c == 0, s < NS - 1))
    def _():
        pltpu.sync_copy(xw_hbm.at[pl.ds(s * RPW, RPW)],
                        agg_sh.at[pl.ds(s * RPW, RPW)])

    @pl.when(jnp.logical_and(c == 0, s == NS - 1))
    def _():
        pltpu.sync_copy(xw_hbm.at[pl.ds((NS - 1) * RPW, lastw)],
                        agg_sh.at[pl.ds((NS - 1) * RPW, lastw)])
        pltpu.sync_copy(zrow_hbm.at[pl.ds(0, NTRASH)],
                        agg_sh.at[pl.ds(N, NTRASH)])

    @pl.when(c == 1)
    def _():
        pltpu.sync_copy(zrow_hbm.at[pl.ds(s * RPW, RPW)],
                        agg_sh.at[pl.ds(s * RPW, RPW)])

    pltpu.sync_copy(zrow_hbm.at[pl.ds(s * RPW, RPW)],
                    cnt_sh.at[pl.ds(s * RPW, RPW)])

    def fill_ones(r, carry):
        ones_v[r] = jnp.ones((H,), jnp.float32)
        return carry

    lax.fori_loop(0, SCH, fill_ones, 0)
    plsc.subcore_barrier()
    pltpu.sync_copy(ei_hbm.at[0, pl.ds(t * EPT, EPT)],
                    src_v.at[pl.ds(0, EPT)])
    pltpu.sync_copy(ei_hbm.at[1, pl.ds(t * EPT, EPT)],
                    dst_v.at[pl.ds(0, EPT)])
    pltpu.sync_copy(psrc_hbm.at[pl.ds(t * PADT, PADT)],
                    src_v.at[pl.ds(EPT, PADT)])
    pltpu.sync_copy(pdst_hbm.at[pl.ds(t * PADT, PADT)],
                    dst_v.at[pl.ds(EPT, PADT)])

    def start_gather(g, b):
        pltpu.async_copy(xw_hbm.at[src_v.at[pl.ds(g * SCH, SCH)]],
                         rows_v.at[b], gsem.at[b])

    def wait_gather(b):
        pltpu.make_async_copy(xw_hbm.at[src_v.at[pl.ds(0, SCH)]],
                              rows_v.at[b], gsem.at[b]).wait()

    def start_scatters(g, b):
        pltpu.async_copy(rows_v.at[b], agg_sh.at[dst_v.at[pl.ds(g * SCH, SCH)]],
                         ssem.at[b], add=True)
        pltpu.async_copy(ones_v, cnt_sh.at[dst_v.at[pl.ds(g * SCH, SCH)]],
                         csem, add=True)

    def wait_scatter(b):
        pltpu.make_async_copy(rows_v.at[b], agg_sh.at[dst_v.at[pl.ds(0, SCH)]],
                              ssem.at[b]).wait()

    for g in range(LOOK):
        start_gather(g, g)
    for g in range(KB):
        b = g % RING
        wait_gather(b)
        start_scatters(g, b)
        nxt = g + LOOK
        if nxt < KB:
            bn = nxt % RING
            if nxt >= RING:
                wait_scatter(bn)
            start_gather(nxt, bn)
    for g in range(KB - RING, KB):
        wait_scatter(g % RING)
    for g in range(KB):
        pltpu.make_async_copy(ones_v, cnt_sh.at[dst_v.at[pl.ds(0, SCH)]],
                              csem).wait()
    plsc.subcore_barrier()
    pltpu.sync_copy(agg_sh.at[pl.ds(s * RPW, RPW)],
                    agg_out.at[c, pl.ds(s * RPW, RPW)])
    pltpu.sync_copy(cnt_sh.at[pl.ds(s * RPW, RPW)],
                    cnt_out.at[c, pl.ds(s * RPW, RPW)])


def _mlp_body(p_ref, c_ref, bias_ref, w1_ref, b1_ref, w2_ref, b2_ref, o_ref):
    s = p_ref[0, :NR, :] + p_ref[1, :NR, :]
    cnt = c_ref[0, :NR, :] + c_ref[1, :NR, :] + 1.0
    h = jnp.maximum(s / cnt + bias_ref[...], 0.0)
    h = jnp.maximum(
        jnp.dot(h, w1_ref[...], preferred_element_type=jnp.float32)
        + b1_ref[...], 0.0)
    y = (jnp.dot(h, w2_ref[...], preferred_element_type=jnp.float32)
         + b2_ref[...])
    o_ref[...] = jax.nn.sigmoid(y)


def _blockdiag(m, k):
    r, ccol = m.shape
    out = jnp.zeros((k, r, k, ccol), m.dtype)
    out = out.at[jnp.arange(k), :, jnp.arange(k), :].set(m)
    return out.reshape(k * r, k * ccol)


def kernel(x, edge_index, W, u, c, bias, W1, b1, W2, b2):
    del u, c
    ei = edge_index.astype(jnp.int32)
    psrc = jnp.zeros((NT * PADT,), jnp.int32)
    pdst = N + (jnp.arange(NT * PADT, dtype=jnp.int32) % NTRASH)

    b4 = jnp.stack([jnp.pad(W, ((0, 0), (a * H, 128 - (a + 1) * H)))
                    for a in range(PK)])
    xw8 = pl.pallas_call(
        _xw_body,
        out_shape=jax.ShapeDtypeStruct((NR, 128), jnp.float32),
    )(x.reshape(NR, PK, D), b4)

    zrow = jnp.zeros((NPAD, H), jnp.float32)
    parts, cnts = _edge_scatter(xw8.reshape(N, H), ei, psrc, pdst, zrow)

    y8 = pl.pallas_call(
        _mlp_body,
        out_shape=jax.ShapeDtypeStruct((NR, PK), jnp.float32),
    )(parts.reshape(NC, NRP, 128), cnts.reshape(NC, NRP, 128),
      jnp.tile(bias, PK).reshape(1, PK * H), _blockdiag(W1, PK),
      jnp.tile(b1, PK).reshape(1, PK * 8), _blockdiag(W2, PK),
      b2.reshape(1, 1))
    return y8.reshape(N, 1)


# speedup vs baseline: 1.9676x; 1.0199x over previous
import functools

import jax
import jax.numpy as jnp
from jax import lax
from jax.experimental import pallas as pl
from jax.experimental.pallas import tpu as pltpu
from jax.experimental.pallas import tpu_sc as plsc

N = 10000
E = 320000
D = 128
H = 16
NC, NS = 2, 16
NT = NC * NS
SCH = 512
KB = 20
EPAD = KB * SCH
RING = 8
LOOK = 3
EPT = E // NT
PADT = EPAD - EPT
NPAD = 10112
RPW = NPAD // NS
PK = 128 // H
NR = N * H // 128
NRP = NPAD * H // 128
NTRASH = NPAD - N


def _xw_body(x3_ref, b_ref, o_ref):
    acc = jnp.dot(x3_ref[:, 0, :], b_ref[0],
                  preferred_element_type=jnp.float32)
    for a in range(1, PK):
        acc += jnp.dot(x3_ref[:, a, :], b_ref[a],
                       preferred_element_type=jnp.float32)
    o_ref[...] = acc


_sc_mesh = plsc.VectorSubcoreMesh(core_axis_name="c", subcore_axis_name="s")


@functools.partial(
    pl.kernel,
    out_type=[
        jax.ShapeDtypeStruct((NC, NPAD, H), jnp.float32),
        jax.ShapeDtypeStruct((NC, NPAD), jnp.float32),
    ],
    mesh=_sc_mesh,
    scratch_types=[
        pltpu.VMEM((EPAD,), jnp.int32),
        pltpu.VMEM((EPAD,), jnp.int32),
        pltpu.VMEM((RING, SCH, H), jnp.float32),
        pltpu.VMEM((SCH,), jnp.float32),
        pltpu.VMEM_SHARED((NPAD, H), jnp.float32),
        pltpu.VMEM_SHARED((NPAD,), jnp.float32),
        pltpu.SemaphoreType.DMA((RING,)),
        pltpu.SemaphoreType.DMA((RING,)),
        pltpu.SemaphoreType.DMA,
    ],
    compiler_params=pltpu.CompilerParams(use_tc_tiling_on_sc=False),
)
def _edge_scatter(xw_hbm, ei_hbm, psrc_hbm, pdst_hbm, zrow_hbm, zcnt_hbm,
                  agg_out, cnt_out,
                  src_v, dst_v, rows_v, ones_v,
                  agg_sh, cnt_sh, gsem, ssem, csem):
    c = lax.axis_index("c")
    s = lax.axis_index("s")
    t = c * NS + s
    lastw = N - (NS - 1) * RPW

    @pl.when(jnp.logical_and(c == 0, s < NS - 1))
    def _():
        pltpu.sync_copy(xw_hbm.at[pl.ds(s * RPW, RPW)],
                        agg_sh.at[pl.ds(s * RPW, RPW)])

    @pl.when(jnp.logical_and(c == 0, s == NS - 1))
    def _():
        pltpu.sync_copy(xw_hbm.at[pl.ds((NS - 1) * RPW, lastw)],
                        agg_sh.at[pl.ds((NS - 1) * RPW, lastw)])
        pltpu.sync_copy(zrow_hbm.at[pl.ds(0, NTRASH)],
                        agg_sh.at[pl.ds(N, NTRASH)])

    @pl.when(c == 1)
    def _():
        pltpu.sync_copy(zrow_hbm.at[pl.ds(s * RPW, RPW)],
                        agg_sh.at[pl.ds(s * RPW, RPW)])

    pltpu.sync_copy(zcnt_hbm.at[pl.ds(s * RPW, RPW)],
                    cnt_sh.at[pl.ds(s * RPW, RPW)])
    for k in range(SCH // 16):
        ones_v[pl.ds(k * 16, 16)] = jnp.ones((16,), jnp.float32)
    plsc.subcore_barrier()
    pltpu.sync_copy(ei_hbm.at[0, pl.ds(t * EPT, EPT)],
                    src_v.at[pl.ds(0, EPT)])
    pltpu.sync_copy(ei_hbm.at[1, pl.ds(t * EPT, EPT)],
                    dst_v.at[pl.ds(0, EPT)])
    pltpu.sync_copy(psrc_hbm.at[pl.ds(t * PADT, PADT)],
                    src_v.at[pl.ds(EPT, PADT)])
    pltpu.sync_copy(pdst_hbm.at[pl.ds(t * PADT, PADT)],
                    dst_v.at[pl.ds(EPT, PADT)])

    def start_gather(g, b):
        pltpu.async_copy(xw_hbm.at[src_v.at[pl.ds(g * SCH, SCH)]],
                         rows_v.at[b], gsem.at[b])

    def wait_gather(b):
        pltpu.make_async_copy(xw_hbm.at[src_v.at[pl.ds(0, SCH)]],
                              rows_v.at[b], gsem.at[b]).wait()

    def start_scatters(g, b):
        pltpu.async_copy(rows_v.at[b], agg_sh.at[dst_v.at[pl.ds(g * SCH, SCH)]],
                         ssem.at[b], add=True)
        pltpu.async_copy(ones_v, cnt_sh.at[dst_v.at[pl.ds(g * SCH, SCH)]],
                         csem, add=True)

    def wait_scatter(b):
        pltpu.make_async_copy(rows_v.at[b], agg_sh.at[dst_v.at[pl.ds(0, SCH)]],
                              ssem.at[b]).wait()

    for g in range(LOOK):
        start_gather(g, g)
    for g in range(KB):
        b = g % RING
        wait_gather(b)
        start_scatters(g, b)
        nxt = g + LOOK
        if nxt < KB:
            bn = nxt % RING
            if nxt >= RING:
                wait_scatter(bn)
            start_gather(nxt, bn)
    for g in range(KB - RING, KB):
        wait_scatter(g % RING)
    for g in range(KB):
        pltpu.make_async_copy(ones_v, cnt_sh.at[dst_v.at[pl.ds(0, SCH)]],
                              csem).wait()
    plsc.subcore_barrier()
    pltpu.sync_copy(agg_sh.at[pl.ds(s * RPW, RPW)],
                    agg_out.at[c, pl.ds(s * RPW, RPW)])
    pltpu.sync_copy(cnt_sh.at[pl.ds(s * RPW, RPW)],
                    cnt_out.at[c, pl.ds(s * RPW, RPW)])


def _mlp_body(p_ref, c_ref, g_ref, bias_ref, w1_ref, b1_ref, w2_ref, b2_ref,
              o_ref):
    s = p_ref[0, :NR, :] + p_ref[1, :NR, :]
    csum = c_ref[0] + c_ref[1]
    blocks = [jnp.dot(csum, g_ref[i], preferred_element_type=jnp.float32)
              for i in range(16)]
    cnt = jnp.stack(blocks, axis=1).reshape(NRP, 128)[:NR, :] + 1.0
    h = jnp.maximum(s / cnt + bias_ref[...], 0.0)
    h = jnp.maximum(
        jnp.dot(h, w1_ref[...], preferred_element_type=jnp.float32)
        + b1_ref[...], 0.0)
    y = (jnp.dot(h, w2_ref[...], preferred_element_type=jnp.float32)
         + b2_ref[...])
    o_ref[...] = jax.nn.sigmoid(y)


def _blockdiag(m, k):
    r, ccol = m.shape
    out = jnp.zeros((k, r, k, ccol), m.dtype)
    out = out.at[jnp.arange(k), :, jnp.arange(k), :].set(m)
    return out.reshape(k * r, k * ccol)


def kernel(x, edge_index, W, u, c, bias, W1, b1, W2, b2):
    del u, c
    ei = edge_index.astype(jnp.int32)
    psrc = jnp.zeros((NT * PADT,), jnp.int32)
    pdst = N + (jnp.arange(NT * PADT, dtype=jnp.int32) % NTRASH)

    b4 = jnp.stack([jnp.pad(W, ((0, 0), (a * H, 128 - (a + 1) * H)))
                    for a in range(PK)])
    xw8 = pl.pallas_call(
        _xw_body,
        out_shape=jax.ShapeDtypeStruct((NR, 128), jnp.float32),
    )(x.reshape(NR, PK, D), b4)

    zrow = jnp.zeros((NPAD, H), jnp.float32)
    zcnt = jnp.zeros((NPAD,), jnp.float32)
    parts, cnts = _edge_scatter(xw8.reshape(N, H), ei, psrc, pdst, zrow, zcnt)

    ii = jnp.arange(16).reshape(16, 1, 1)
    aa = jnp.arange(PK).reshape(1, PK, 1)
    hh = jnp.arange(H).reshape(1, 1, H)
    g = jnp.zeros((16, 128, 128), jnp.float32).at[
        jnp.broadcast_to(ii, (16, PK, H)),
        jnp.broadcast_to(8 * ii + aa, (16, PK, H)),
        jnp.broadcast_to(aa * H + hh, (16, PK, H))].set(1.0)

    y8 = pl.pallas_call(
        _mlp_body,
        out_shape=jax.ShapeDtypeStruct((NR, PK), jnp.float32),
    )(parts.reshape(NC, NRP, 128), cnts.reshape(NC, NPAD // 128, 128), g,
      jnp.tile(bias, PK).reshape(1, PK * H), _blockdiag(W1, PK),
      jnp.tile(b1, PK).reshape(1, PK * 8), _blockdiag(W2, PK),
      b2.reshape(1, 1))
    return y8.reshape(N, 1)


# speedup vs baseline: 1.9756x; 1.0041x over previous
import functools

import jax
import jax.numpy as jnp
from jax import lax
from jax.experimental import pallas as pl
from jax.experimental.pallas import tpu as pltpu
from jax.experimental.pallas import tpu_sc as plsc

N = 10000
E = 320000
D = 128
H = 16
NC, NS = 2, 16
NT = NC * NS
SCH = 512
KB = 20
EPAD = KB * SCH
RING = 8
LOOK = 3
EPT = E // NT
PADT = EPAD - EPT
NPAD = 10112
RPW = NPAD // NS
PK = 128 // H
NR = N * H // 128
NRP = NPAD * H // 128
NTRASH = NPAD - N


def _xw_body(x_ref, b_ref, o_ref):
    x3 = x_ref[...].reshape(NR, PK, D)
    acc = jnp.dot(x3[:, 0, :], b_ref[0],
                  preferred_element_type=jnp.float32)
    for a in range(1, PK):
        acc += jnp.dot(x3[:, a, :], b_ref[a],
                       preferred_element_type=jnp.float32)
    o_ref[...] = acc


_sc_mesh = plsc.VectorSubcoreMesh(core_axis_name="c", subcore_axis_name="s")


@functools.partial(
    pl.kernel,
    out_type=[
        jax.ShapeDtypeStruct((NC, NPAD, H), jnp.float32),
        jax.ShapeDtypeStruct((NC, NPAD), jnp.float32),
    ],
    mesh=_sc_mesh,
    scratch_types=[
        pltpu.VMEM((EPAD,), jnp.int32),
        pltpu.VMEM((EPAD,), jnp.int32),
        pltpu.VMEM((RING, SCH, H), jnp.float32),
        pltpu.VMEM((SCH,), jnp.float32),
        pltpu.VMEM_SHARED((NPAD, H), jnp.float32),
        pltpu.VMEM_SHARED((NPAD,), jnp.float32),
        pltpu.SemaphoreType.DMA((RING,)),
        pltpu.SemaphoreType.DMA((RING,)),
        pltpu.SemaphoreType.DMA,
    ],
    compiler_params=pltpu.CompilerParams(use_tc_tiling_on_sc=False),
)
def _edge_scatter(xw_hbm, ei_hbm, psrc_hbm, pdst_hbm, zrow_hbm, zcnt_hbm,
                  agg_out, cnt_out,
                  src_v, dst_v, rows_v, ones_v,
                  agg_sh, cnt_sh, gsem, ssem, csem):
    c = lax.axis_index("c")
    s = lax.axis_index("s")
    t = c * NS + s
    lastw = N - (NS - 1) * RPW

    @pl.when(jnp.logical_and(c == 0, s < NS - 1))
    def _():
        pltpu.sync_copy(xw_hbm.at[pl.ds(s * RPW, RPW)],
                        agg_sh.at[pl.ds(s * RPW, RPW)])

    @pl.when(jnp.logical_and(c == 0, s == NS - 1))
    def _():
        pltpu.sync_copy(xw_hbm.at[pl.ds((NS - 1) * RPW, lastw)],
                        agg_sh.at[pl.ds((NS - 1) * RPW, lastw)])
        pltpu.sync_copy(zrow_hbm.at[pl.ds(0, NTRASH)],
                        agg_sh.at[pl.ds(N, NTRASH)])

    @pl.when(c == 1)
    def _():
        pltpu.sync_copy(zrow_hbm.at[pl.ds(s * RPW, RPW)],
                        agg_sh.at[pl.ds(s * RPW, RPW)])

    pltpu.sync_copy(zcnt_hbm.at[pl.ds(s * RPW, RPW)],
                    cnt_sh.at[pl.ds(s * RPW, RPW)])
    for k in range(SCH // 16):
        ones_v[pl.ds(k * 16, 16)] = jnp.ones((16,), jnp.float32)
    plsc.subcore_barrier()
    pltpu.sync_copy(ei_hbm.at[0, pl.ds(t * EPT, EPT)],
                    src_v.at[pl.ds(0, EPT)])
    pltpu.sync_copy(ei_hbm.at[1, pl.ds(t * EPT, EPT)],
                    dst_v.at[pl.ds(0, EPT)])
    pltpu.sync_copy(psrc_hbm.at[pl.ds(t * PADT, PADT)],
                    src_v.at[pl.ds(EPT, PADT)])
    pltpu.sync_copy(pdst_hbm.at[pl.ds(t * PADT, PADT)],
                    dst_v.at[pl.ds(EPT, PADT)])

    def start_gather(g, b):
        pltpu.async_copy(xw_hbm.at[src_v.at[pl.ds(g * SCH, SCH)]],
                         rows_v.at[b], gsem.at[b])

    def wait_gather(b):
        pltpu.make_async_copy(xw_hbm.at[src_v.at[pl.ds(0, SCH)]],
                              rows_v.at[b], gsem.at[b]).wait()

    def start_scatters(g, b):
        pltpu.async_copy(rows_v.at[b], agg_sh.at[dst_v.at[pl.ds(g * SCH, SCH)]],
                         ssem.at[b], add=True)
        pltpu.async_copy(ones_v, cnt_sh.at[dst_v.at[pl.ds(g * SCH, SCH)]],
                         csem, add=True)

    def wait_scatter(b):
        pltpu.make_async_copy(rows_v.at[b], agg_sh.at[dst_v.at[pl.ds(0, SCH)]],
                              ssem.at[b]).wait()

    for g in range(LOOK):
        start_gather(g, g)
    for g in range(KB):
        b = g % RING
        wait_gather(b)
        start_scatters(g, b)
        nxt = g + LOOK
        if nxt < KB:
            bn = nxt % RING
            if nxt >= RING:
                wait_scatter(bn)
            start_gather(nxt, bn)
    for g in range(KB - RING, KB):
        wait_scatter(g % RING)
    for g in range(KB):
        pltpu.make_async_copy(ones_v, cnt_sh.at[dst_v.at[pl.ds(0, SCH)]],
                              csem).wait()
    plsc.subcore_barrier()
    pltpu.sync_copy(agg_sh.at[pl.ds(s * RPW, RPW)],
                    agg_out.at[c, pl.ds(s * RPW, RPW)])
    pltpu.sync_copy(cnt_sh.at[pl.ds(s * RPW, RPW)],
                    cnt_out.at[c, pl.ds(s * RPW, RPW)])


def _mlp_body(p_ref, c_ref, g_ref, bias_ref, w1_ref, b1_ref, w2_ref, b2_ref,
              o_ref):
    s = p_ref[0, :NR, :] + p_ref[1, :NR, :]
    csum = c_ref[0] + c_ref[1]
    blocks = [jnp.dot(csum, g_ref[i], preferred_element_type=jnp.float32)
              for i in range(16)]
    cnt = jnp.stack(blocks, axis=1).reshape(NRP, 128)[:NR, :] + 1.0
    h = jnp.maximum(s / cnt + bias_ref[...], 0.0)
    h = jnp.maximum(
        jnp.dot(h, w1_ref[...], preferred_element_type=jnp.float32)
        + b1_ref[...], 0.0)
    y = (jnp.dot(h, w2_ref[...], preferred_element_type=jnp.float32)
         + b2_ref[...])
    o_ref[...] = jax.nn.sigmoid(y)


def _blockdiag(m, k):
    r, ccol = m.shape
    out = jnp.zeros((k, r, k, ccol), m.dtype)
    out = out.at[jnp.arange(k), :, jnp.arange(k), :].set(m)
    return out.reshape(k * r, k * ccol)


def kernel(x, edge_index, W, u, c, bias, W1, b1, W2, b2):
    del u, c
    ei = edge_index.astype(jnp.int32)
    psrc = jnp.zeros((NT * PADT,), jnp.int32)
    pdst = N + (jnp.arange(NT * PADT, dtype=jnp.int32) % NTRASH)

    b4 = jnp.stack([jnp.pad(W, ((0, 0), (a * H, 128 - (a + 1) * H)))
                    for a in range(PK)])
    xw8 = pl.pallas_call(
        _xw_body,
        out_shape=jax.ShapeDtypeStruct((NR, 128), jnp.float32),
    )(x, b4)

    zrow = jnp.zeros((NPAD, H), jnp.float32)
    zcnt = jnp.zeros((NPAD,), jnp.float32)
    parts, cnts = _edge_scatter(xw8.reshape(N, H), ei, psrc, pdst, zrow, zcnt)

    ii = jnp.arange(16).reshape(16, 1, 1)
    aa = jnp.arange(PK).reshape(1, PK, 1)
    hh = jnp.arange(H).reshape(1, 1, H)
    g = jnp.zeros((16, 128, 128), jnp.float32).at[
        jnp.broadcast_to(ii, (16, PK, H)),
        jnp.broadcast_to(8 * ii + aa, (16, PK, H)),
        jnp.broadcast_to(aa * H + hh, (16, PK, H))].set(1.0)

    y8 = pl.pallas_call(
        _mlp_body,
        out_shape=jax.ShapeDtypeStruct((NR, PK), jnp.float32),
    )(parts.reshape(NC, NRP, 128), cnts.reshape(NC, NPAD // 128, 128), g,
      jnp.tile(bias, PK).reshape(1, PK * H), _blockdiag(W1, PK),
      jnp.tile(b1, PK).reshape(1, PK * 8), _blockdiag(W2, PK),
      b2.reshape(1, 1))
    return y8.reshape(N, 1)
